# baseline (speedup 1.0000x reference)
import functools

import jax
import jax.numpy as jnp
from jax import lax
from jax.experimental import pallas as pl
from jax.experimental.pallas import tpu as pltpu
from jax.experimental.pallas import tpu_sc as plsc

_VOCAB = 1000000
_D = 64
_BATCH = 4096
_HIST = 200

_NC, _NS = 2, 16
_NW = _NC * _NS
_RPW = _BATCH // _NW
_NBUF = 4
_NGROUPS = _HIST // _NBUF

_mesh = plsc.VectorSubcoreMesh(
    core_axis_name="c", subcore_axis_name="s",
    num_cores=_NC, num_subcores=_NS)


@functools.partial(
    pl.kernel,
    out_type=jax.ShapeDtypeStruct((_BATCH, _HIST, _D), jnp.float32),
    mesh=_mesh,
    scratch_types=[
        pltpu.VMEM((_HIST, _RPW), jnp.int32),
        pltpu.VMEM((_NBUF, _RPW, _D), jnp.float32),
    ] + [pltpu.SemaphoreType.DMA] * _NBUF,
    compiler_params=pltpu.CompilerParams(use_tc_tiling_on_sc=False),
)
def _emb_gather(xt_hbm, table_hbm, out_hbm, idx_v, rows_v, s0, s1, s2, s3):
    sems = (s0, s1, s2, s3)
    wid = lax.axis_index("s") * _NC + lax.axis_index("c")
    row0 = wid * _RPW

    pltpu.sync_copy(xt_hbm.at[:, pl.ds(row0, _RPW)], idx_v)

    def gather_desc(t, b):
        return pltpu.make_async_copy(
            table_hbm.at[idx_v.at[t]], rows_v.at[b], sems[b])

    for b in range(_NBUF):
        gather_desc(b, b).start()

    def group(g, carry):
        for b in range(_NBUF):
            t = g * _NBUF + b
            gather_desc(t, b).wait()
            pltpu.sync_copy(rows_v.at[b], out_hbm.at[pl.ds(row0, _RPW), t])
            nt = t + _NBUF

            @pl.when(nt < _HIST)
            def _():
                gather_desc(nt, b).start()
        return carry

    lax.fori_loop(0, _NGROUPS, group, 0)


def kernel(x, table):
    return _emb_gather(x.astype(jnp.int32).T, table)
